# Initial kernel scaffold; baseline (speedup 1.0000x reference)
import jax
import jax.numpy as jnp
from jax.experimental import pallas as pl


def kernel(x, prefixes, suffixes, word_emb, prefix_emb, suffix_emb, W1, b1, W2, b2):
    raise NotImplementedError("write your pallas kernel here")


# speedup vs baseline: 1.4320x; 1.4320x over previous
import jax
import jax.numpy as jnp
from jax import lax
from jax.experimental import pallas as pl
from jax.experimental.pallas import tpu as pltpu
from jax.experimental.pallas import tpu_sc as plsc

EMB_DIM = 50
EMB_PAD = 56
WINDOW = 5
HIDDEN = 512
OUT = 45
BATCH = 16384

NUM_WORKERS = 32
TOTAL_ROWS = BATCH * WINDOW
ROWS_PER_W = TOTAL_ROWS // NUM_WORKERS
CHUNK = 128
NCHUNK = ROWS_PER_W // CHUNK

BM = 1024
KP = EMB_PAD * WINDOW


def _sc_gather_sum_body(wt, pt, iw, ip, isf, out,
                        xw, xp, xs, bw, bp, bs, ob, sw, sp, ss):
  wid = lax.axis_index("s") * 2 + lax.axis_index("c")
  base0 = wid * ROWS_PER_W

  def chunk(j, carry):
    base = base0 + j * CHUNK
    pltpu.sync_copy(iw.at[pl.ds(base, CHUNK)], xw)
    pltpu.sync_copy(ip.at[pl.ds(base, CHUNK)], xp)
    pltpu.sync_copy(isf.at[pl.ds(base, CHUNK)], xs)
    cw = pltpu.async_copy(wt.at[xw], bw, sw)
    cp = pltpu.async_copy(pt.at[xp], bp, sp)
    cs = pltpu.async_copy(pt.at[xs], bs, ss)
    cw.wait()
    cp.wait()
    cs.wait()

    def row(r, carry2):
      for c in (0, 16, 32, 40):
        ob[r, pl.ds(c, 16)] = (bw[r, pl.ds(c, 16)] + bp[r, pl.ds(c, 16)]
                               + bs[r, pl.ds(c, 16)])
      return carry2

    lax.fori_loop(0, CHUNK, row, 0, unroll=False)
    pltpu.sync_copy(ob, out.at[pl.ds(base, CHUNK)])
    return carry

  lax.fori_loop(0, NCHUNK, chunk, 0, unroll=False)


_sc_gather_sum = pl.kernel(
    _sc_gather_sum_body,
    out_type=jax.ShapeDtypeStruct((TOTAL_ROWS, EMB_PAD), jnp.float32),
    mesh=plsc.VectorSubcoreMesh(core_axis_name="c", subcore_axis_name="s"),
    scratch_types=[
        pltpu.VMEM((CHUNK,), jnp.int32),
        pltpu.VMEM((CHUNK,), jnp.int32),
        pltpu.VMEM((CHUNK,), jnp.int32),
        pltpu.VMEM((CHUNK, EMB_PAD), jnp.float32),
        pltpu.VMEM((CHUNK, EMB_PAD), jnp.float32),
        pltpu.VMEM((CHUNK, EMB_PAD), jnp.float32),
        pltpu.VMEM((CHUNK, EMB_PAD), jnp.float32),
        pltpu.SemaphoreType.DMA,
        pltpu.SemaphoreType.DMA,
        pltpu.SemaphoreType.DMA,
    ],
    compiler_params=pltpu.CompilerParams(use_tc_tiling_on_sc=False),
)


def _mlp_body(e_ref, w1_ref, b1_ref, w2_ref, b2_ref, o_ref):
  e = e_ref[...]
  h = jnp.tanh(
      jnp.dot(e, w1_ref[...], preferred_element_type=jnp.float32)
      + b1_ref[...])
  lg = (jnp.dot(h, w2_ref[...], preferred_element_type=jnp.float32)
        + b2_ref[...])
  m = jnp.max(lg, axis=-1, keepdims=True)
  s = lg - m
  o_ref[...] = s - jnp.log(jnp.sum(jnp.exp(s), axis=-1, keepdims=True))


def _mlp(e, w1p, b1, w2t, b2):
  return pl.pallas_call(
      _mlp_body,
      grid=(BATCH // BM,),
      in_specs=[
          pl.BlockSpec((BM, KP), lambda i: (i, 0)),
          pl.BlockSpec((KP, HIDDEN), lambda i: (0, 0)),
          pl.BlockSpec((1, HIDDEN), lambda i: (0, 0)),
          pl.BlockSpec((HIDDEN, OUT), lambda i: (0, 0)),
          pl.BlockSpec((1, OUT), lambda i: (0, 0)),
      ],
      out_specs=pl.BlockSpec((BM, OUT), lambda i: (i, 0)),
      out_shape=jax.ShapeDtypeStruct((BATCH, OUT), jnp.float32),
  )(e, w1p, b1, w2t, b2)


@jax.jit
def kernel(x, prefixes, suffixes, word_emb, prefix_emb, suffix_emb,
           W1, b1, W2, b2):
  del suffix_emb
  iw = x.astype(jnp.int32).reshape(-1)
  ip = prefixes.astype(jnp.int32).reshape(-1)
  isf = suffixes.astype(jnp.int32).reshape(-1)
  wtp = jnp.pad(word_emb, ((0, 0), (0, EMB_PAD - EMB_DIM)))
  ptp = jnp.pad(prefix_emb, ((0, 0), (0, EMB_PAD - EMB_DIM)))
  esum = _sc_gather_sum(wtp, ptp, iw, ip, isf)
  e = esum.reshape(BATCH, KP)
  w1p = jnp.pad(W1.reshape(HIDDEN, WINDOW, EMB_DIM),
                ((0, 0), (0, 0), (0, EMB_PAD - EMB_DIM))).reshape(HIDDEN, KP).T
  return _mlp(e, w1p, b1.reshape(1, HIDDEN), W2.T, b2.reshape(1, OUT))
